# Initial kernel scaffold; baseline (speedup 1.0000x reference)
import functools

import jax
import jax.numpy as jnp
from jax import lax
from jax.experimental import pallas as pl
from jax.experimental.pallas import tpu as pltpu
from jax.experimental.pallas import tpu_sc as plsc

N = 10000
D = 128
DA = 144
E = 320000
NC, NS = 2, 16
NW = NC * NS
CH = 128
NCH = 79
E_PAD = NW * NCH * CH
N_ACC = 10240
N_DUMMY = N_ACC - N
ROWS_PER_TILE = N_ACC // NS


def _sc_gather_scatter(x_aug, row_r, col_r):
    mesh = plsc.VectorSubcoreMesh(core_axis_name="c", subcore_axis_name="s")

    @functools.partial(
        pl.kernel,
        out_type=jax.ShapeDtypeStruct((NC, N_ACC, DA), jnp.float32),
        mesh=mesh,
        scratch_types=[
            pltpu.VMEM((NCH, 1, CH), jnp.int32),
            pltpu.VMEM((NCH, 1, CH), jnp.int32),
            pltpu.VMEM((CH, DA), jnp.float32),
            pltpu.VMEM_SHARED((N_ACC, DA), jnp.float32),
            pltpu.SemaphoreType.DMA,
        ],
    )
    def k(x_hbm, row_hbm, col_hbm, out_hbm, row_v, col_v, rows_v, acc, sem):
        core = lax.axis_index("c")
        sub = lax.axis_index("s")
        wid = core * NS + sub
        base = sub * ROWS_PER_TILE

        zeros16 = jnp.zeros((16,), jnp.float32)

        @pl.loop(0, CH)
        def _(r):
            @pl.loop(0, DA // 16)
            def _(j):
                rows_v[r, pl.ds(j * 16, 16)] = zeros16

        @pl.loop(0, ROWS_PER_TILE // CH)
        def _(b):
            pltpu.sync_copy(rows_v, acc.at[pl.ds(base + b * CH, CH)])

        pltpu.sync_copy(row_hbm.at[wid], row_v)
        pltpu.sync_copy(col_hbm.at[wid], col_v)

        plsc.subcore_barrier()

        @pl.loop(0, NCH)
        def _(c):
            pltpu.async_copy(x_hbm.at[col_v.at[c, 0]], rows_v, sem).wait()
            pltpu.sync_copy(rows_v, acc.at[row_v.at[c, 0]], add=True)

        plsc.subcore_barrier()

        pltpu.sync_copy(acc.at[pl.ds(base, ROWS_PER_TILE)],
                        out_hbm.at[core, pl.ds(base, ROWS_PER_TILE)])

    return k(x_aug, row_r, col_r)


def _tc_finish(partials, x, wn_t, wr_t, bias2):
    blk = 1000

    def body(p_ref, x_ref, wn_ref, wr_ref, b_ref, o_ref):
        s = p_ref[0] + p_ref[1]
        cnt = jnp.maximum(s[:, D:D + 1], 1.0)
        aggr = s[:, :D] / cnt
        out = jnp.dot(aggr, wn_ref[...], preferred_element_type=jnp.float32)
        out = out + jnp.dot(x_ref[...], wr_ref[...],
                            preferred_element_type=jnp.float32)
        o_ref[...] = out + b_ref[...]

    return pl.pallas_call(
        body,
        grid=(N // blk,),
        in_specs=[
            pl.BlockSpec((NC, blk, DA), lambda i: (0, i, 0)),
            pl.BlockSpec((blk, D), lambda i: (i, 0)),
            pl.BlockSpec((D, D), lambda i: (0, 0)),
            pl.BlockSpec((D, D), lambda i: (0, 0)),
            pl.BlockSpec((1, D), lambda i: (0, 0)),
        ],
        out_specs=pl.BlockSpec((blk, D), lambda i: (i, 0)),
        out_shape=jax.ShapeDtypeStruct((N, D), jnp.float32),
    )(partials, x, wn_t, wr_t, bias2)


def kernel(x, edge_index, W_neigh, W_root, bias):
    row = edge_index[0].astype(jnp.int32)
    col = edge_index[1].astype(jnp.int32)
    pad = E_PAD - E
    ar = jnp.arange(pad, dtype=jnp.int32)
    row_p = jnp.concatenate([row, N + ar % N_DUMMY])
    col_p = jnp.concatenate([col, ar % N])
    row_r = row_p.reshape(NW, NCH, 1, CH)
    col_r = col_p.reshape(NW, NCH, 1, CH)
    x_aug = jnp.concatenate(
        [x, jnp.ones((N, 1), jnp.float32),
         jnp.zeros((N, DA - D - 1), jnp.float32)], axis=1)
    partials = _sc_gather_scatter(x_aug, row_r, col_r)
    return _tc_finish(partials, x, W_neigh.T, W_root.T, bias.reshape(1, D))


# speedup vs baseline: 7.8628x; 7.8628x over previous
import functools

import jax
import jax.numpy as jnp
from jax import lax
from jax.experimental import pallas as pl
from jax.experimental.pallas import tpu as pltpu
from jax.experimental.pallas import tpu_sc as plsc

N = 10000
D = 128
DA = 144
E = 320000
NC, NS = 2, 16
NW = NC * NS
CH = 64
NCH = 160
E_PAD = NW * NCH * CH
N_ACC = 10112
N_DUMMY = N_ACC - N
ROWS_PER_TILE = N_ACC // NS


def _sc_gather_scatter(x_aug, row_r, col_r):
    mesh = plsc.VectorSubcoreMesh(core_axis_name="c", subcore_axis_name="s")

    @functools.partial(
        pl.kernel,
        out_type=jax.ShapeDtypeStruct((NC, N_ACC, DA), jnp.float32),
        mesh=mesh,
        scratch_types=[
            pltpu.VMEM((NCH, 1, CH), jnp.int32),
            pltpu.VMEM((NCH, 1, CH), jnp.int32),
            pltpu.VMEM((CH, DA), jnp.float32),
            pltpu.VMEM((CH, DA), jnp.float32),
            pltpu.VMEM_SHARED((N_ACC, DA), jnp.float32),
            pltpu.SemaphoreType.DMA,
            pltpu.SemaphoreType.DMA,
        ],
        compiler_params=pltpu.CompilerParams(use_tc_tiling_on_sc=False),
    )
    def k(x_hbm, row_hbm, col_hbm, out_hbm,
          row_v, col_v, buf0, buf1, acc, sem0, sem1):
        core = lax.axis_index("c")
        sub = lax.axis_index("s")
        wid = core * NS + sub
        base = sub * ROWS_PER_TILE

        zeros16 = jnp.zeros((16,), jnp.float32)

        @pl.loop(0, CH)
        def _(r):
            @pl.loop(0, DA // 16)
            def _(j):
                buf0[r, pl.ds(j * 16, 16)] = zeros16

        @pl.loop(0, ROWS_PER_TILE // CH)
        def _(b):
            pltpu.sync_copy(buf0, acc.at[pl.ds(base + b * CH, CH)])

        rem = ROWS_PER_TILE % CH
        if rem:
            pltpu.sync_copy(
                buf0.at[pl.ds(0, rem)],
                acc.at[pl.ds(base + ROWS_PER_TILE - rem, rem)])

        pltpu.sync_copy(row_hbm.at[wid], row_v)
        pltpu.sync_copy(col_hbm.at[wid], col_v)

        plsc.subcore_barrier()

        @pl.loop(0, NCH, step=2)
        def _(c):
            a0 = pltpu.async_copy(x_hbm.at[col_v.at[c, 0]], buf0, sem0)
            a1 = pltpu.async_copy(x_hbm.at[col_v.at[c + 1, 0]], buf1, sem1)
            a0.wait()
            pltpu.sync_copy(buf0, acc.at[row_v.at[c, 0]], add=True)
            a1.wait()
            pltpu.sync_copy(buf1, acc.at[row_v.at[c + 1, 0]], add=True)

        plsc.subcore_barrier()

        pltpu.sync_copy(acc.at[pl.ds(base, ROWS_PER_TILE)],
                        out_hbm.at[core, pl.ds(base, ROWS_PER_TILE)])

    return k(x_aug, row_r, col_r)


def _tc_finish(partials, x, wn_t, wr_t, bias2):
    blk = 1000

    def body(p_ref, x_ref, wn_ref, wr_ref, b_ref, o_ref):
        s = p_ref[0] + p_ref[1]
        cnt = jnp.maximum(s[:, D:D + 1], 1.0)
        aggr = s[:, :D] / cnt
        out = jnp.dot(aggr, wn_ref[...], preferred_element_type=jnp.float32)
        out = out + jnp.dot(x_ref[...], wr_ref[...],
                            preferred_element_type=jnp.float32)
        o_ref[...] = out + b_ref[...]

    return pl.pallas_call(
        body,
        grid=(N // blk,),
        in_specs=[
            pl.BlockSpec((NC, blk, DA), lambda i: (0, i, 0)),
            pl.BlockSpec((blk, D), lambda i: (i, 0)),
            pl.BlockSpec((D, D), lambda i: (0, 0)),
            pl.BlockSpec((D, D), lambda i: (0, 0)),
            pl.BlockSpec((1, D), lambda i: (0, 0)),
        ],
        out_specs=pl.BlockSpec((blk, D), lambda i: (i, 0)),
        out_shape=jax.ShapeDtypeStruct((N, D), jnp.float32),
    )(partials, x, wn_t, wr_t, bias2)


def kernel(x, edge_index, W_neigh, W_root, bias):
    row = edge_index[0].astype(jnp.int32)
    col = edge_index[1].astype(jnp.int32)
    pad = E_PAD - E
    ar = jnp.arange(pad, dtype=jnp.int32)
    row_p = jnp.concatenate([row, N + ar % N_DUMMY])
    col_p = jnp.concatenate([col, ar % N])
    row_r = row_p.reshape(NW, NCH, 1, CH)
    col_r = col_p.reshape(NW, NCH, 1, CH)
    x_aug = jnp.concatenate(
        [x, jnp.ones((N, 1), jnp.float32),
         jnp.zeros((N, DA - D - 1), jnp.float32)], axis=1)
    partials = _sc_gather_scatter(x_aug, row_r, col_r)
    return _tc_finish(partials, x, W_neigh.T, W_root.T, bias.reshape(1, D))


# speedup vs baseline: 7.8738x; 1.0014x over previous
import functools

import jax
import jax.numpy as jnp
from jax import lax
from jax.experimental import pallas as pl
from jax.experimental.pallas import tpu as pltpu
from jax.experimental.pallas import tpu_sc as plsc

N = 10000
D = 128
DA = 144
E = 320000
NC, NS = 2, 16
NW = NC * NS
CH = 64
NCH = 160
E_PAD = NW * NCH * CH
N_ACC = 10112
N_DUMMY = N_ACC - N
ROWS_PER_TILE = N_ACC // NS


def _sc_gather_scatter(x_aug, row_r, col_r):
    mesh = plsc.VectorSubcoreMesh(core_axis_name="c", subcore_axis_name="s")

    @functools.partial(
        pl.kernel,
        out_type=jax.ShapeDtypeStruct((NC, N_ACC, DA), jnp.float32),
        mesh=mesh,
        scratch_types=[
            pltpu.VMEM((NCH, 1, CH), jnp.int32),
            pltpu.VMEM((NCH, 1, CH), jnp.int32),
            pltpu.VMEM((CH, DA), jnp.float32),
            pltpu.VMEM((CH, DA), jnp.float32),
            pltpu.VMEM_SHARED((N_ACC, DA), jnp.float32),
            pltpu.SemaphoreType.DMA,
            pltpu.SemaphoreType.DMA,
        ],
        compiler_params=pltpu.CompilerParams(use_tc_tiling_on_sc=False,
                                             disable_bounds_checks=True),
    )
    def k(x_hbm, row_hbm, col_hbm, out_hbm,
          row_v, col_v, buf0, buf1, acc, sem0, sem1):
        core = lax.axis_index("c")
        sub = lax.axis_index("s")
        wid = core * NS + sub
        base = sub * ROWS_PER_TILE

        zeros16 = jnp.zeros((16,), jnp.float32)

        @pl.loop(0, CH)
        def _(r):
            @pl.loop(0, DA // 16)
            def _(j):
                buf0[r, pl.ds(j * 16, 16)] = zeros16

        @pl.loop(0, ROWS_PER_TILE // CH)
        def _(b):
            pltpu.sync_copy(buf0, acc.at[pl.ds(base + b * CH, CH)])

        rem = ROWS_PER_TILE % CH
        if rem:
            pltpu.sync_copy(
                buf0.at[pl.ds(0, rem)],
                acc.at[pl.ds(base + ROWS_PER_TILE - rem, rem)])

        pltpu.sync_copy(row_hbm.at[wid], row_v)
        pltpu.sync_copy(col_hbm.at[wid], col_v)

        plsc.subcore_barrier()

        @pl.loop(0, NCH, step=2)
        def _(c):
            a0 = pltpu.async_copy(x_hbm.at[col_v.at[c, 0]], buf0, sem0)
            a1 = pltpu.async_copy(x_hbm.at[col_v.at[c + 1, 0]], buf1, sem1)
            a0.wait()
            pltpu.sync_copy(buf0, acc.at[row_v.at[c, 0]], add=True)
            a1.wait()
            pltpu.sync_copy(buf1, acc.at[row_v.at[c + 1, 0]], add=True)

        plsc.subcore_barrier()

        pltpu.sync_copy(acc.at[pl.ds(base, ROWS_PER_TILE)],
                        out_hbm.at[core, pl.ds(base, ROWS_PER_TILE)])

    return k(x_aug, row_r, col_r)


def _tc_finish(partials, x, wn_t, wr_t, bias2):
    blk = 1000

    def body(p_ref, x_ref, wn_ref, wr_ref, b_ref, o_ref):
        s = p_ref[0] + p_ref[1]
        cnt = jnp.maximum(s[:, D:D + 1], 1.0)
        aggr = s[:, :D] / cnt
        out = jnp.dot(aggr, wn_ref[...], preferred_element_type=jnp.float32)
        out = out + jnp.dot(x_ref[...], wr_ref[...],
                            preferred_element_type=jnp.float32)
        o_ref[...] = out + b_ref[...]

    return pl.pallas_call(
        body,
        grid=(N // blk,),
        in_specs=[
            pl.BlockSpec((NC, blk, DA), lambda i: (0, i, 0)),
            pl.BlockSpec((blk, D), lambda i: (i, 0)),
            pl.BlockSpec((D, D), lambda i: (0, 0)),
            pl.BlockSpec((D, D), lambda i: (0, 0)),
            pl.BlockSpec((1, D), lambda i: (0, 0)),
        ],
        out_specs=pl.BlockSpec((blk, D), lambda i: (i, 0)),
        out_shape=jax.ShapeDtypeStruct((N, D), jnp.float32),
    )(partials, x, wn_t, wr_t, bias2)


def kernel(x, edge_index, W_neigh, W_root, bias):
    row = edge_index[0].astype(jnp.int32)
    col = edge_index[1].astype(jnp.int32)
    pad = E_PAD - E
    ar = jnp.arange(pad, dtype=jnp.int32)
    row_p = jnp.concatenate([row, N + ar % N_DUMMY])
    col_p = jnp.concatenate([col, ar % N])
    row_r = row_p.reshape(NW, NCH, 1, CH)
    col_r = col_p.reshape(NW, NCH, 1, CH)
    x_aug = jnp.concatenate(
        [x, jnp.ones((N, 1), jnp.float32),
         jnp.zeros((N, DA - D - 1), jnp.float32)], axis=1)
    partials = _sc_gather_scatter(x_aug, row_r, col_r)
    return _tc_finish(partials, x, W_neigh.T, W_root.T, bias.reshape(1, D))


# speedup vs baseline: 8.6824x; 1.1027x over previous
import functools

import jax
import jax.numpy as jnp
from jax import lax
from jax.experimental import pallas as pl
from jax.experimental.pallas import tpu as pltpu
from jax.experimental.pallas import tpu_sc as plsc

N = 10000
D = 128
E = 320000
CNTW = 16
NC, NS = 2, 16
NW = NC * NS
CH = 40
NCH = 250
ROWS_PER_TILE = N // NS


def _sc_gather_scatter(x, e_r):
    mesh = plsc.VectorSubcoreMesh(core_axis_name="c", subcore_axis_name="s")

    @functools.partial(
        pl.kernel,
        out_type=(jax.ShapeDtypeStruct((NC, N, D), jnp.float32),
                  jax.ShapeDtypeStruct((NC, N, CNTW), jnp.float32)),
        mesh=mesh,
        scratch_types=[
            pltpu.VMEM((NCH, 1, CH), jnp.int32),
            pltpu.VMEM((NCH, 1, CH), jnp.int32),
            pltpu.VMEM((CH, D), jnp.float32),
            pltpu.VMEM((CH, D), jnp.float32),
            pltpu.VMEM((CH, CNTW), jnp.float32),
            pltpu.VMEM((CH, CNTW), jnp.float32),
            pltpu.VMEM_SHARED((N, D), jnp.float32),
            pltpu.VMEM_SHARED((N, CNTW), jnp.float32),
            pltpu.SemaphoreType.DMA,
            pltpu.SemaphoreType.DMA,
        ],
        compiler_params=pltpu.CompilerParams(use_tc_tiling_on_sc=False,
                                             disable_bounds_checks=True),
    )
    def k(x_hbm, e_hbm, outf_hbm, outc_hbm,
          row_v, col_v, buf0, buf1, ones_v, zer_v, accf, accc, sem0, sem1):
        core = lax.axis_index("c")
        sub = lax.axis_index("s")
        wid = core * NS + sub
        base = sub * ROWS_PER_TILE

        zeros16 = jnp.zeros((16,), jnp.float32)
        ones16 = jnp.ones((16,), jnp.float32)

        @pl.loop(0, CH)
        def _(r):
            ones_v[r, pl.ds(0, 16)] = ones16
            zer_v[r, pl.ds(0, 16)] = zeros16

            @pl.loop(0, D // 16)
            def _(j):
                buf0[r, pl.ds(j * 16, 16)] = zeros16

        @pl.loop(0, ROWS_PER_TILE // CH)
        def _(b):
            pltpu.sync_copy(buf0, accf.at[pl.ds(base + b * CH, CH)])
            pltpu.sync_copy(zer_v, accc.at[pl.ds(base + b * CH, CH)])

        rem = ROWS_PER_TILE % CH
        if rem:
            off = base + ROWS_PER_TILE - rem
            pltpu.sync_copy(buf0.at[pl.ds(0, rem)], accf.at[pl.ds(off, rem)])
            pltpu.sync_copy(zer_v.at[pl.ds(0, rem)], accc.at[pl.ds(off, rem)])

        pltpu.sync_copy(e_hbm.at[0, wid], row_v)
        pltpu.sync_copy(e_hbm.at[1, wid], col_v)

        plsc.subcore_barrier()

        @pl.loop(0, NCH, step=2)
        def _(c):
            a0 = pltpu.async_copy(x_hbm.at[col_v.at[c, 0]], buf0, sem0)
            a1 = pltpu.async_copy(x_hbm.at[col_v.at[c + 1, 0]], buf1, sem1)
            a0.wait()
            pltpu.sync_copy(buf0, accf.at[row_v.at[c, 0]], add=True)
            pltpu.sync_copy(ones_v, accc.at[row_v.at[c, 0]], add=True)
            a1.wait()
            pltpu.sync_copy(buf1, accf.at[row_v.at[c + 1, 0]], add=True)
            pltpu.sync_copy(ones_v, accc.at[row_v.at[c + 1, 0]], add=True)

        plsc.subcore_barrier()

        pltpu.sync_copy(accf.at[pl.ds(base, ROWS_PER_TILE)],
                        outf_hbm.at[core, pl.ds(base, ROWS_PER_TILE)])
        pltpu.sync_copy(accc.at[pl.ds(base, ROWS_PER_TILE)],
                        outc_hbm.at[core, pl.ds(base, ROWS_PER_TILE)])

    return k(x, e_r)


def _tc_finish(pf, cnt, x, wn_t, wr_t, bias2):
    blk = 1000

    def body(p_ref, c_ref, x_ref, wn_ref, wr_ref, b_ref, o_ref):
        s = p_ref[0] + p_ref[1]
        cnt = jnp.maximum(c_ref[...], 1.0)
        aggr = s / cnt
        out = jnp.dot(aggr, wn_ref[...], preferred_element_type=jnp.float32)
        out = out + jnp.dot(x_ref[...], wr_ref[...],
                            preferred_element_type=jnp.float32)
        o_ref[...] = out + b_ref[...]

    return pl.pallas_call(
        body,
        grid=(N // blk,),
        in_specs=[
            pl.BlockSpec((NC, blk, D), lambda i: (0, i, 0)),
            pl.BlockSpec((blk, 1), lambda i: (i, 0)),
            pl.BlockSpec((blk, D), lambda i: (i, 0)),
            pl.BlockSpec((D, D), lambda i: (0, 0)),
            pl.BlockSpec((D, D), lambda i: (0, 0)),
            pl.BlockSpec((1, D), lambda i: (0, 0)),
        ],
        out_specs=pl.BlockSpec((blk, D), lambda i: (i, 0)),
        out_shape=jax.ShapeDtypeStruct((N, D), jnp.float32),
    )(pf, cnt, x, wn_t, wr_t, bias2)


def kernel(x, edge_index, W_neigh, W_root, bias):
    e_r = edge_index.astype(jnp.int32).reshape(2, NW, NCH, 1, CH)
    pf, pc = _sc_gather_scatter(x, e_r)
    cnt = (pc[0] + pc[1])[:, :1]
    return _tc_finish(pf, cnt, x, W_neigh.T, W_root.T, bias.reshape(1, D))


# speedup vs baseline: 11.0532x; 1.2731x over previous
import functools

import jax
import jax.numpy as jnp
from jax import lax
from jax.experimental import pallas as pl
from jax.experimental.pallas import tpu as pltpu
from jax.experimental.pallas import tpu_sc as plsc

N = 10000
D = 128
E = 320000
CNTW = 16
N_CNT = 10048
NC, NS = 2, 16
NW = NC * NS
CH = 40
NCH = 250
ROWS_PER_TILE = N // NS
CNT_ROWS_PER_TILE = N_CNT // NS


def _sc_gather_scatter(x, e_r):
    mesh = plsc.VectorSubcoreMesh(core_axis_name="c", subcore_axis_name="s")

    @functools.partial(
        pl.kernel,
        out_type=(jax.ShapeDtypeStruct((NC, N, D), jnp.float32),
                  jax.ShapeDtypeStruct((NC, N_CNT, CNTW), jnp.float32)),
        mesh=mesh,
        scratch_types=[
            pltpu.VMEM((NCH, 1, CH), jnp.int32),
            pltpu.VMEM((NCH, 1, CH), jnp.int32),
            pltpu.VMEM((CH, D), jnp.float32),
            pltpu.VMEM((CH, D), jnp.float32),
            pltpu.VMEM((CH, CNTW), jnp.float32),
            pltpu.VMEM((CH, CNTW), jnp.float32),
            pltpu.VMEM_SHARED((N, D), jnp.float32),
            pltpu.VMEM_SHARED((N_CNT, CNTW), jnp.float32),
            pltpu.SemaphoreType.DMA,
            pltpu.SemaphoreType.DMA,
        ],
        compiler_params=pltpu.CompilerParams(use_tc_tiling_on_sc=False,
                                             disable_bounds_checks=True),
    )
    def k(x_hbm, e_hbm, outf_hbm, outc_hbm,
          row_v, col_v, buf0, buf1, ones_v, zer_v, accf, accc, sem0, sem1):
        core = lax.axis_index("c")
        sub = lax.axis_index("s")
        wid = core * NS + sub
        base = sub * ROWS_PER_TILE
        cbase = sub * CNT_ROWS_PER_TILE

        zeros16 = jnp.zeros((16,), jnp.float32)
        ones16 = jnp.ones((16,), jnp.float32)

        @pl.loop(0, CH)
        def _(r):
            ones_v[r, pl.ds(0, 16)] = ones16
            zer_v[r, pl.ds(0, 16)] = zeros16

            @pl.loop(0, D // 16)
            def _(j):
                buf0[r, pl.ds(j * 16, 16)] = zeros16

        @pl.loop(0, ROWS_PER_TILE // CH)
        def _(b):
            pltpu.sync_copy(buf0, accf.at[pl.ds(base + b * CH, CH)])

        rem = ROWS_PER_TILE % CH
        if rem:
            pltpu.sync_copy(buf0.at[pl.ds(0, rem)],
                            accf.at[pl.ds(base + ROWS_PER_TILE - rem, rem)])

        @pl.loop(0, CNT_ROWS_PER_TILE // CH)
        def _(b):
            pltpu.sync_copy(zer_v, accc.at[pl.ds(cbase + b * CH, CH)])

        crem = CNT_ROWS_PER_TILE % CH
        if crem:
            pltpu.sync_copy(
                zer_v.at[pl.ds(0, crem)],
                accc.at[pl.ds(cbase + CNT_ROWS_PER_TILE - crem, crem)])

        pltpu.sync_copy(e_hbm.at[0, wid], row_v)
        pltpu.sync_copy(e_hbm.at[1, wid], col_v)

        plsc.subcore_barrier()

        pltpu.async_copy(x_hbm.at[col_v.at[0, 0]], buf0, sem0)
        pltpu.async_copy(x_hbm.at[col_v.at[1, 0]], buf1, sem1)

        @pl.loop(0, NCH, step=2)
        def _(c):
            pltpu.make_async_copy(x_hbm.at[col_v.at[c, 0]], buf0, sem0).wait()
            pltpu.sync_copy(buf0, accf.at[row_v.at[c, 0]], add=True)
            c2 = jnp.minimum(c + 2, NCH - 2)
            pltpu.async_copy(x_hbm.at[col_v.at[c2, 0]], buf0, sem0)
            pltpu.sync_copy(ones_v, accc.at[row_v.at[c, 0]], add=True)

            pltpu.make_async_copy(x_hbm.at[col_v.at[c + 1, 0]], buf1,
                                  sem1).wait()
            pltpu.sync_copy(buf1, accf.at[row_v.at[c + 1, 0]], add=True)
            c3 = jnp.minimum(c + 3, NCH - 1)
            pltpu.async_copy(x_hbm.at[col_v.at[c3, 0]], buf1, sem1)
            pltpu.sync_copy(ones_v, accc.at[row_v.at[c + 1, 0]], add=True)

        pltpu.make_async_copy(x_hbm.at[col_v.at[NCH - 2, 0]], buf0,
                              sem0).wait()
        pltpu.make_async_copy(x_hbm.at[col_v.at[NCH - 1, 0]], buf1,
                              sem1).wait()

        plsc.subcore_barrier()

        pltpu.sync_copy(accf.at[pl.ds(base, ROWS_PER_TILE)],
                        outf_hbm.at[core, pl.ds(base, ROWS_PER_TILE)])
        pltpu.sync_copy(accc.at[pl.ds(cbase, CNT_ROWS_PER_TILE)],
                        outc_hbm.at[core, pl.ds(cbase, CNT_ROWS_PER_TILE)])

    return k(x, e_r)


def _tc_finish(pf, cnt, x, wn_t, wr_t, bias2):
    blk = 1000

    def body(p_ref, c_ref, x_ref, wn_ref, wr_ref, b_ref, o_ref):
        s = p_ref[0] + p_ref[1]
        aggr = s / jnp.maximum(c_ref[...], 1.0)
        out = jnp.dot(aggr, wn_ref[...], preferred_element_type=jnp.float32)
        out = out + jnp.dot(x_ref[...], wr_ref[...],
                            preferred_element_type=jnp.float32)
        o_ref[...] = out + b_ref[...]

    return pl.pallas_call(
        body,
        grid=(N // blk,),
        in_specs=[
            pl.BlockSpec((NC, blk, D), lambda i: (0, i, 0)),
            pl.BlockSpec((blk, 1), lambda i: (i, 0)),
            pl.BlockSpec((blk, D), lambda i: (i, 0)),
            pl.BlockSpec((D, D), lambda i: (0, 0)),
            pl.BlockSpec((D, D), lambda i: (0, 0)),
            pl.BlockSpec((1, D), lambda i: (0, 0)),
        ],
        out_specs=pl.BlockSpec((blk, D), lambda i: (i, 0)),
        out_shape=jax.ShapeDtypeStruct((N, D), jnp.float32),
    )(pf, cnt, x, wn_t, wr_t, bias2)


def kernel(x, edge_index, W_neigh, W_root, bias):
    e_r = edge_index.astype(jnp.int32).reshape(2, NW, NCH, 1, CH)
    pf, pc = _sc_gather_scatter(x, e_r)
    cnt = (pc[0] + pc[1])[:N, :1]
    return _tc_finish(pf, cnt, x, W_neigh.T, W_root.T, bias.reshape(1, D))


# speedup vs baseline: 12.3503x; 1.1174x over previous
import functools

import jax
import jax.numpy as jnp
from jax import lax
from jax.experimental import pallas as pl
from jax.experimental.pallas import tpu as pltpu
from jax.experimental.pallas import tpu_sc as plsc

N = 10000
D = 128
E = 320000
CNTW = 16
N_CNT = 10048
NC, NS = 2, 16
NW = NC * NS
CH = 40
NCH = 250
ROWS_PER_TILE = N // NS
CNT_ROWS_PER_TILE = N_CNT // NS


def _sc_gather_scatter(x, e_r):
    mesh = plsc.VectorSubcoreMesh(core_axis_name="c", subcore_axis_name="s")

    @functools.partial(
        pl.kernel,
        out_type=(jax.ShapeDtypeStruct((NC, N, D), jnp.float32),
                  jax.ShapeDtypeStruct((NC, N_CNT, CNTW), jnp.float32)),
        mesh=mesh,
        scratch_types=[
            pltpu.VMEM((NCH, 1, CH), jnp.int32),
            pltpu.VMEM((NCH, 1, CH), jnp.int32),
            pltpu.VMEM((CH, D), jnp.float32),
            pltpu.VMEM((CH, D), jnp.float32),
            pltpu.VMEM((CH, D), jnp.float32),
            pltpu.VMEM((CH, CNTW), jnp.float32),
            pltpu.VMEM((CH, CNTW), jnp.float32),
            pltpu.VMEM_SHARED((N, D), jnp.float32),
            pltpu.VMEM_SHARED((N_CNT, CNTW), jnp.float32),
            pltpu.SemaphoreType.DMA,
            pltpu.SemaphoreType.DMA,
            pltpu.SemaphoreType.DMA,
            pltpu.SemaphoreType.DMA,
            pltpu.SemaphoreType.DMA,
            pltpu.SemaphoreType.DMA,
            pltpu.SemaphoreType.DMA,
        ],
        compiler_params=pltpu.CompilerParams(use_tc_tiling_on_sc=False,
                                             disable_bounds_checks=True),
    )
    def k(x_hbm, e_hbm, outf_hbm, outc_hbm,
          row_v, col_v, buf0, buf1, buf2, ones_v, zer_v, accf, accc,
          gs0, gs1, gs2, fs0, fs1, fs2, cs):
        core = lax.axis_index("c")
        sub = lax.axis_index("s")
        wid = core * NS + sub
        base = sub * ROWS_PER_TILE
        cbase = sub * CNT_ROWS_PER_TILE

        zeros16 = jnp.zeros((16,), jnp.float32)
        ones16 = jnp.ones((16,), jnp.float32)

        @pl.loop(0, CH)
        def _(r):
            ones_v[r, pl.ds(0, 16)] = ones16
            zer_v[r, pl.ds(0, 16)] = zeros16

            @pl.loop(0, D // 16)
            def _(j):
                buf0[r, pl.ds(j * 16, 16)] = zeros16

        @pl.loop(0, ROWS_PER_TILE // CH)
        def _(b):
            pltpu.sync_copy(buf0, accf.at[pl.ds(base + b * CH, CH)])

        rem = ROWS_PER_TILE % CH
        if rem:
            pltpu.sync_copy(buf0.at[pl.ds(0, rem)],
                            accf.at[pl.ds(base + ROWS_PER_TILE - rem, rem)])

        @pl.loop(0, CNT_ROWS_PER_TILE // CH)
        def _(b):
            pltpu.sync_copy(zer_v, accc.at[pl.ds(cbase + b * CH, CH)])

        crem = CNT_ROWS_PER_TILE % CH
        if crem:
            pltpu.sync_copy(
                zer_v.at[pl.ds(0, crem)],
                accc.at[pl.ds(cbase + CNT_ROWS_PER_TILE - crem, crem)])

        pltpu.sync_copy(e_hbm.at[0, wid], row_v)
        pltpu.sync_copy(e_hbm.at[1, wid], col_v)

        plsc.subcore_barrier()

        pltpu.async_copy(x_hbm.at[col_v.at[0, 0]], buf0, gs0)
        pltpu.async_copy(x_hbm.at[col_v.at[1, 0]], buf1, gs1)
        pltpu.async_copy(x_hbm.at[col_v.at[2, 0]], buf2, gs2)

        @pl.loop(0, NCH - 1, step=3)
        def _(c):
            pltpu.make_async_copy(x_hbm.at[col_v.at[c, 0]], buf0, gs0).wait()
            f0 = pltpu.async_copy(buf0, accf.at[row_v.at[c, 0]], fs0,
                                  add=True)
            pltpu.make_async_copy(x_hbm.at[col_v.at[c + 1, 0]], buf1,
                                  gs1).wait()
            f1 = pltpu.async_copy(buf1, accf.at[row_v.at[c + 1, 0]], fs1,
                                  add=True)
            pltpu.make_async_copy(x_hbm.at[col_v.at[c + 2, 0]], buf2,
                                  gs2).wait()
            f2 = pltpu.async_copy(buf2, accf.at[row_v.at[c + 2, 0]], fs2,
                                  add=True)
            cc0 = pltpu.async_copy(ones_v, accc.at[row_v.at[c, 0]], cs,
                                   add=True)
            cc1 = pltpu.async_copy(ones_v, accc.at[row_v.at[c + 1, 0]], cs,
                                   add=True)
            cc2 = pltpu.async_copy(ones_v, accc.at[row_v.at[c + 2, 0]], cs,
                                   add=True)
            f0.wait()
            pltpu.async_copy(x_hbm.at[col_v.at[c + 3, 0]], buf0, gs0)
            f1.wait()
            c4 = jnp.minimum(c + 4, NCH - 1)
            pltpu.async_copy(x_hbm.at[col_v.at[c4, 0]], buf1, gs1)
            f2.wait()
            c5 = jnp.minimum(c + 5, NCH - 1)
            pltpu.async_copy(x_hbm.at[col_v.at[c5, 0]], buf2, gs2)
            cc0.wait()
            cc1.wait()
            cc2.wait()

        pltpu.make_async_copy(x_hbm.at[col_v.at[NCH - 1, 0]], buf0,
                              gs0).wait()
        pltpu.sync_copy(buf0, accf.at[row_v.at[NCH - 1, 0]], add=True)
        pltpu.sync_copy(ones_v, accc.at[row_v.at[NCH - 1, 0]], add=True)
        pltpu.make_async_copy(x_hbm.at[col_v.at[NCH - 1, 0]], buf1,
                              gs1).wait()
        pltpu.make_async_copy(x_hbm.at[col_v.at[NCH - 1, 0]], buf2,
                              gs2).wait()

        plsc.subcore_barrier()

        pltpu.sync_copy(accf.at[pl.ds(base, ROWS_PER_TILE)],
                        outf_hbm.at[core, pl.ds(base, ROWS_PER_TILE)])
        pltpu.sync_copy(accc.at[pl.ds(cbase, CNT_ROWS_PER_TILE)],
                        outc_hbm.at[core, pl.ds(cbase, CNT_ROWS_PER_TILE)])

    return k(x, e_r)


def _tc_finish(pf, cnt, x, wn_t, wr_t, bias2):
    blk = 1000

    def body(p_ref, c_ref, x_ref, wn_ref, wr_ref, b_ref, o_ref):
        s = p_ref[0] + p_ref[1]
        aggr = s / jnp.maximum(c_ref[...], 1.0)
        out = jnp.dot(aggr, wn_ref[...], preferred_element_type=jnp.float32)
        out = out + jnp.dot(x_ref[...], wr_ref[...],
                            preferred_element_type=jnp.float32)
        o_ref[...] = out + b_ref[...]

    return pl.pallas_call(
        body,
        grid=(N // blk,),
        in_specs=[
            pl.BlockSpec((NC, blk, D), lambda i: (0, i, 0)),
            pl.BlockSpec((blk, 1), lambda i: (i, 0)),
            pl.BlockSpec((blk, D), lambda i: (i, 0)),
            pl.BlockSpec((D, D), lambda i: (0, 0)),
            pl.BlockSpec((D, D), lambda i: (0, 0)),
            pl.BlockSpec((1, D), lambda i: (0, 0)),
        ],
        out_specs=pl.BlockSpec((blk, D), lambda i: (i, 0)),
        out_shape=jax.ShapeDtypeStruct((N, D), jnp.float32),
    )(pf, cnt, x, wn_t, wr_t, bias2)


def kernel(x, edge_index, W_neigh, W_root, bias):
    e_r = edge_index.astype(jnp.int32).reshape(2, NW, NCH, 1, CH)
    pf, pc = _sc_gather_scatter(x, e_r)
    cnt = (pc[0] + pc[1])[:N, :1]
    return _tc_finish(pf, cnt, x, W_neigh.T, W_root.T, bias.reshape(1, D))
